# baseline (speedup 1.0000x reference)
import functools

import jax
import jax.numpy as jnp
from jax import lax
from jax.experimental import pallas as pl
from jax.experimental.pallas import tpu as pltpu
from jax.experimental.pallas import tpu_sc as plsc

N_NODES = 10000
NC, NS, L = 2, 16, 16
NW = NC * NS
CHUNK = 128
NB = 4
ROWS_PER_TILE = 632
NPAD = NS * ROWS_PER_TILE

_MESH = plsc.VectorSubcoreMesh(
    core_axis_name="c", subcore_axis_name="s", num_cores=NC, num_subcores=NS
)
_SC_PARAMS = pltpu.CompilerParams(
    needs_layout_passes=False, use_tc_tiling_on_sc=False
)

_CORE0_FRAC = 0.3


def _split(per_tile):
    total = 2 * per_tile
    m0 = int(round(total * _CORE0_FRAC / 8.0)) * 8
    m0 = min(max(m0, 8), total - 8)
    return m0, total - m0


def _deg_body(m0, m1, dst2, degp, deg_v, didx2, si):
    c_ax = lax.axis_index("c")
    s_ax = lax.axis_index("s")
    wid = c_ax * NS + s_ax
    mm = max(m0, m1)
    is0 = c_ax == 0
    m = jnp.where(is0, m0, m1)
    base = jnp.where(is0, s_ax * m0, NS * m0 + s_ax * m1)

    islab = pl.multiple_of(base, 8)
    cp = pltpu.async_copy(dst2.at[pl.ds(islab, mm)], didx2, si)

    def zero(r, carry):
        deg_v[pl.ds(r * L, L)] = jnp.zeros((L,), jnp.float32)
        return carry

    lax.fori_loop(0, NPAD // L, zero, 0)
    cp.wait()

    ones = jnp.ones((L,), jnp.float32)

    def body(c, carry):
        for k in range(CHUNK // L):
            idx = didx2[c, pl.ds(k * L, L)]
            plsc.addupdate_scatter(deg_v, [idx], ones)
        return carry

    lax.fori_loop(0, m, body, 0)
    out_off = pl.multiple_of(wid * NPAD, 8)
    pltpu.sync_copy(deg_v, degp.at[pl.ds(out_off, NPAD)])


def _make_deg_call(m0, m1):
    return pl.kernel(
        functools.partial(_deg_body, m0, m1),
        out_type=jax.ShapeDtypeStruct((NW * NPAD,), jnp.float32),
        mesh=_MESH,
        scratch_types=[
            pltpu.VMEM((NPAD,), jnp.float32),
            pltpu.VMEM((max(m0, m1), CHUNK), jnp.int32),
            pltpu.SemaphoreType.DMA,
        ],
        compiler_params=_SC_PARAMS,
    )


def _agg_body(m0, m1, feat, g_hbm, src2, dst2, out_hbm,
              acc_sh, sidx2, didx2, rows0, rows1, rows2, rows3,
              si, sg0, sg1, sg2, sg3, ss0, ss1, ss2, ss3):
    rows = (rows0, rows1, rows2, rows3)
    sg = (sg0, sg1, sg2, sg3)
    ss = (ss0, ss1, ss2, ss3)
    c_ax = lax.axis_index("c")
    s_ax = lax.axis_index("s")
    mm = max(m0, m1)
    is0 = c_ax == 0
    m = jnp.where(is0, m0, m1)
    base = jnp.where(is0, s_ax * m0, NS * m0 + s_ax * m1)

    islab = pl.multiple_of(base, 8)
    cp_s = pltpu.async_copy(src2.at[pl.ds(islab, mm)], sidx2, si)
    cp_d = pltpu.async_copy(dst2.at[pl.ds(islab, mm)], didx2, si)

    def zero(r, carry):
        for k in range(feat // L):
            rows0[r, pl.ds(k * L, L)] = jnp.zeros((L,), jnp.float32)
        return carry

    lax.fori_loop(0, CHUNK, zero, 0)
    slab = pl.multiple_of(s_ax * ROWS_PER_TILE, 8)
    n_full, rem = ROWS_PER_TILE // CHUNK, ROWS_PER_TILE % CHUNK
    for q in range(n_full):
        pltpu.sync_copy(rows0, acc_sh.at[pl.ds(slab + q * CHUNK, CHUNK)])
    if rem:
        pltpu.sync_copy(
            rows0.at[pl.ds(0, rem)],
            acc_sh.at[pl.ds(slab + n_full * CHUNK, rem)],
        )
    cp_s.wait()
    cp_d.wait()
    plsc.subcore_barrier()

    def gather_start(c, b):
        pltpu.async_copy(g_hbm.at[sidx2.at[c]], rows[b], sg[b])

    def gather_wait(b):
        pltpu.make_async_copy(g_hbm.at[sidx2.at[0]], rows[b], sg[b]).wait()

    def scatter_start(c, b):
        pltpu.async_copy(rows[b], acc_sh.at[didx2.at[c]], ss[b], add=True)

    def scatter_wait(b):
        pltpu.make_async_copy(rows[b], acc_sh.at[pl.ds(0, CHUNK)], ss[b]).wait()

    def outer(t, carry):
        for b in range(NB):
            c = t * NB + b
            pb = (b + NB - 1) % NB

            @pl.when(t > 0)
            def _():
                scatter_wait(b)

            gather_start(c, b)
            if b == 0:
                @pl.when(t > 0)
                def _():
                    gather_wait(pb)
                    scatter_start(t * NB - 1, pb)
            else:
                gather_wait(pb)
                scatter_start(c - 1, pb)
        return carry

    lax.fori_loop(0, m // NB, outer, 0)

    gather_wait(NB - 1)
    scatter_start(m - 1, NB - 1)
    for b in range(NB):
        scatter_wait(b)
    plsc.subcore_barrier()
    pltpu.sync_copy(acc_sh.at[pl.ds(slab, ROWS_PER_TILE)], out_hbm.at[c_ax, s_ax])


def _make_agg_call(m0, m1, feat):
    mm = max(m0, m1)
    return pl.kernel(
        functools.partial(_agg_body, m0, m1, feat),
        out_type=jax.ShapeDtypeStruct((NC, NS, ROWS_PER_TILE, feat), jnp.float32),
        mesh=_MESH,
        scratch_types=[
            pltpu.VMEM_SHARED((NPAD, feat), jnp.float32),
            pltpu.VMEM((mm, CHUNK), jnp.int32),
            pltpu.VMEM((mm, CHUNK), jnp.int32),
            pltpu.VMEM((CHUNK, feat), jnp.float32),
            pltpu.VMEM((CHUNK, feat), jnp.float32),
            pltpu.VMEM((CHUNK, feat), jnp.float32),
            pltpu.VMEM((CHUNK, feat), jnp.float32),
        ] + [pltpu.SemaphoreType.DMA] * 9,
        compiler_params=_SC_PARAMS,
    )


def _tc1_body(x_ref, w1_ref, degt_ref, g1_ref, dinv_ref):
    deg = jnp.sum(degt_ref[...], axis=1, keepdims=True) + 1.0
    dinv = lax.rsqrt(deg)
    h = jnp.dot(x_ref[...], w1_ref[...], preferred_element_type=jnp.float32)
    g1_ref[...] = h * dinv
    dinv_ref[...] = dinv


def _tc2_body(agg_ref, g1_ref, dinv_ref, b1_ref, w2_ref, g2_ref):
    y = agg_ref[0] + agg_ref[1] + g1_ref[...]
    t = jnp.maximum(dinv_ref[...] * y + b1_ref[...], 0.0)
    h2 = jnp.dot(t, w2_ref[...], preferred_element_type=jnp.float32)
    g2_ref[...] = h2 * dinv_ref[...]


def _tc3_body(agg_ref, g2_ref, dinv_ref, b2_ref, z_ref):
    z = dinv_ref[...] * (agg_ref[0] + agg_ref[1] + g2_ref[...]) + b2_ref[...]
    z_ref[...] = z


def kernel(x, edge_index, W1, b1, W2, b2):
    n_edges = edge_index.shape[1]
    hidden = W1.shape[1]
    z_dim = W2.shape[1]
    per_tile = -(-n_edges // (NW * CHUNK * 8)) * 8
    m0, m1 = _split(per_tile)
    n_rows = NS * (m0 + m1) + max(0, m0 - m1)
    e_pad = n_rows * CHUNK

    src = edge_index[0]
    dst = edge_index[1]
    pad_s = jnp.zeros((e_pad - n_edges,), jnp.int32)
    pad_d = jnp.full((e_pad - n_edges,), N_NODES, jnp.int32)
    src2 = jnp.concatenate([src, pad_s]).reshape(n_rows, CHUNK)
    dst2 = jnp.concatenate([dst, pad_d]).reshape(n_rows, CHUNK)

    degp = _make_deg_call(m0, m1)(dst2)
    degt = degp.reshape(NW, NPAD)[:, :N_NODES].T

    g1, dinv = pl.pallas_call(
        _tc1_body,
        out_shape=(
            jax.ShapeDtypeStruct((N_NODES, hidden), jnp.float32),
            jax.ShapeDtypeStruct((N_NODES, 1), jnp.float32),
        ),
    )(x, W1, degt)

    agg1 = _make_agg_call(m0, m1, hidden)(g1, src2, dst2)
    agg1 = agg1.reshape(NC, NPAD, hidden)[:, :N_NODES]

    g2 = pl.pallas_call(
        _tc2_body,
        out_shape=jax.ShapeDtypeStruct((N_NODES, z_dim), jnp.float32),
    )(agg1, g1, dinv, b1.reshape(1, hidden), W2)

    agg2 = _make_agg_call(m0, m1, z_dim)(g2, src2, dst2)
    agg2 = agg2.reshape(NC, NPAD, z_dim)[:, :N_NODES]

    z = pl.pallas_call(
        _tc3_body,
        out_shape=jax.ShapeDtypeStruct((N_NODES, z_dim), jnp.float32),
    )(agg2, g2, dinv, b2.reshape(1, z_dim))

    return z


# speedup vs baseline: 1.0810x; 1.0810x over previous
import functools

import numpy as np
import jax
import jax.numpy as jnp
from jax import lax
from jax.experimental import pallas as pl
from jax.experimental.pallas import tpu as pltpu
from jax.experimental.pallas import tpu_sc as plsc

N_NODES = 10000
NC, NS, L = 2, 16, 16
NW = NC * NS
CHUNK = 128
NB = 4
ROWS_PER_TILE = 632
NPAD = NS * ROWS_PER_TILE

_MESH = plsc.VectorSubcoreMesh(
    core_axis_name="c", subcore_axis_name="s", num_cores=NC, num_subcores=NS
)
_SC_PARAMS = pltpu.CompilerParams(
    needs_layout_passes=False, use_tc_tiling_on_sc=False
)

_CORE0_FRAC = 0.5


def _split(per_tile):
    total = 2 * per_tile
    m0 = int(round(total * _CORE0_FRAC / float(NB))) * NB
    m0 = min(max(m0, NB), total - NB)
    return m0, total - m0


def _interleave_cols(feat):
    border = []
    for g in range(feat // 32):
        for j in range(16):
            border.append(32 * g + j)
            border.append(32 * g + 16 + j)
    return np.asarray(border, dtype=np.int32)


def _tile_coords():
    c_ax = lax.axis_index("c")
    s_ax = lax.axis_index("s")
    return c_ax, s_ax


def _deg_body(m0, m1, dst2, degp, deg_v, didx2, si):
    c_ax, s_ax = _tile_coords()
    wid = c_ax * NS + s_ax
    mm = max(m0, m1)
    is0 = c_ax == 0
    m = jnp.where(is0, m0, m1)
    base = jnp.where(is0, s_ax * m0, NS * m0 + s_ax * m1)

    cp = pltpu.async_copy(dst2.at[pl.ds(pl.multiple_of(base, NB), mm)], didx2, si)

    def zero(r, carry):
        deg_v[pl.ds(r * L, L)] = jnp.zeros((L,), jnp.float32)
        return carry

    lax.fori_loop(0, NPAD // L, zero, 0)
    cp.wait()

    ones = jnp.ones((L,), jnp.float32)

    def body(c, carry):
        for k in range(CHUNK // L):
            idx = didx2[c, pl.ds(k * L, L)]
            plsc.addupdate_scatter(deg_v, [idx], ones)
        return carry

    lax.fori_loop(0, m, body, 0)
    out_off = pl.multiple_of(wid * NPAD, 8)
    pltpu.sync_copy(deg_v, degp.at[pl.ds(out_off, NPAD)])


def _make_deg_call(m0, m1):
    return pl.kernel(
        functools.partial(_deg_body, m0, m1),
        out_type=jax.ShapeDtypeStruct((NW * NPAD,), jnp.float32),
        mesh=_MESH,
        scratch_types=[
            pltpu.VMEM((NPAD,), jnp.float32),
            pltpu.VMEM((max(m0, m1), CHUNK), jnp.int32),
            pltpu.SemaphoreType.DMA,
        ],
        compiler_params=_SC_PARAMS,
    )


def _agg_body(m0, m1, feat, g_hbm, src2, dst2, out_hbm,
              acc_sh, sidx2, didx2,
              rb0, rb1, rb2, rb3, rf0, rf1, rf2, rf3,
              si, sg0, sg1, sg2, sg3, ss0, ss1, ss2, ss3):
    rows_bf = (rb0, rb1, rb2, rb3)
    rows_f = (rf0, rf1, rf2, rf3)
    sg = (sg0, sg1, sg2, sg3)
    ss = (ss0, ss1, ss2, ss3)
    c_ax, s_ax = _tile_coords()
    mm = max(m0, m1)
    is0 = c_ax == 0
    m = jnp.where(is0, m0, m1)
    base = jnp.where(is0, s_ax * m0, NS * m0 + s_ax * m1)

    islab = pl.multiple_of(base, NB)
    cp_s = pltpu.async_copy(src2.at[pl.ds(islab, mm)], sidx2, si)
    cp_d = pltpu.async_copy(dst2.at[pl.ds(islab, mm)], didx2, si)

    def zero(r, carry):
        for k in range(feat // L):
            rf0[r, pl.ds(k * L, L)] = jnp.zeros((L,), jnp.float32)
        return carry

    lax.fori_loop(0, CHUNK, zero, 0)
    slab = pl.multiple_of(s_ax * ROWS_PER_TILE, 8)
    n_full, rem = ROWS_PER_TILE // CHUNK, ROWS_PER_TILE % CHUNK
    for q in range(n_full):
        pltpu.sync_copy(rf0, acc_sh.at[pl.ds(slab + q * CHUNK, CHUNK)])
    if rem:
        pltpu.sync_copy(
            rf0.at[pl.ds(0, rem)],
            acc_sh.at[pl.ds(slab + n_full * CHUNK, rem)],
        )
    cp_s.wait()
    cp_d.wait()
    plsc.subcore_barrier()

    hi_mask = jnp.full((L,), -65536, jnp.int32)

    def gather_start(c, b):
        pltpu.async_copy(g_hbm.at[sidx2.at[c]], rows_bf[b], sg[b])

    def gather_wait(b):
        pltpu.make_async_copy(g_hbm.at[sidx2.at[0]], rows_bf[b], sg[b]).wait()

    def expand(b):
        src = rows_bf[b]
        dstf = rows_f[b]

        def erow(r, carry):
            for g in range(feat // 32):
                v = plsc.bitcast(src[r, pl.ds(32 * g, 32)], jnp.int32)
                lo = plsc.bitcast(v << 16, jnp.float32)
                hi = plsc.bitcast(v & hi_mask, jnp.float32)
                dstf[r, pl.ds(32 * g, L)] = lo
                dstf[r, pl.ds(32 * g + L, L)] = hi
            return carry

        lax.fori_loop(0, CHUNK, erow, 0)

    def scatter_start(c, b):
        pltpu.async_copy(rows_f[b], acc_sh.at[didx2.at[c]], ss[b], add=True)

    def scatter_wait(b):
        pltpu.make_async_copy(rows_f[b], acc_sh.at[pl.ds(0, CHUNK)], ss[b]).wait()

    def outer(t, carry):
        for b in range(NB):
            c = t * NB + b
            pb = (b + NB - 1) % NB

            @pl.when(t > 0)
            def _():
                scatter_wait(b)

            gather_start(c, b)
            if b == 0:
                @pl.when(t > 0)
                def _():
                    gather_wait(pb)
                    expand(pb)
                    scatter_start(t * NB - 1, pb)
            else:
                gather_wait(pb)
                expand(pb)
                scatter_start(c - 1, pb)
        return carry

    lax.fori_loop(0, m // NB, outer, 0)

    gather_wait(NB - 1)
    expand(NB - 1)
    scatter_start(m - 1, NB - 1)
    for b in range(NB):
        scatter_wait(b)
    plsc.subcore_barrier()
    pltpu.sync_copy(acc_sh.at[pl.ds(slab, ROWS_PER_TILE)], out_hbm.at[c_ax, s_ax])


def _make_agg_call(m0, m1, feat):
    mm = max(m0, m1)
    return pl.kernel(
        functools.partial(_agg_body, m0, m1, feat),
        out_type=jax.ShapeDtypeStruct((NC, NS, ROWS_PER_TILE, feat), jnp.float32),
        mesh=_MESH,
        scratch_types=[
            pltpu.VMEM_SHARED((NPAD, feat), jnp.float32),
            pltpu.VMEM((mm, CHUNK), jnp.int32),
            pltpu.VMEM((mm, CHUNK), jnp.int32),
            pltpu.VMEM((CHUNK, feat), jnp.bfloat16),
            pltpu.VMEM((CHUNK, feat), jnp.bfloat16),
            pltpu.VMEM((CHUNK, feat), jnp.bfloat16),
            pltpu.VMEM((CHUNK, feat), jnp.bfloat16),
            pltpu.VMEM((CHUNK, feat), jnp.float32),
            pltpu.VMEM((CHUNK, feat), jnp.float32),
            pltpu.VMEM((CHUNK, feat), jnp.float32),
            pltpu.VMEM((CHUNK, feat), jnp.float32),
        ] + [pltpu.SemaphoreType.DMA] * 9,
        compiler_params=_SC_PARAMS,
    )


def _tc1_body(x_ref, w1_ref, degt_ref, g1q_ref, dinv_ref):
    deg = jnp.sum(degt_ref[...], axis=1, keepdims=True)
    dinv = lax.rsqrt(jnp.maximum(deg, 1e-12))
    h = jnp.dot(x_ref[...], w1_ref[...], preferred_element_type=jnp.float32)
    g1q_ref[...] = (h * dinv).astype(jnp.bfloat16)
    dinv_ref[...] = dinv


def _tc2_body(agg_ref, dinv_ref, b1_ref, w2_ref, g2q_ref):
    y = agg_ref[0] + agg_ref[1]
    t = jnp.maximum(dinv_ref[...] * y + b1_ref[...], 0.0)
    h2 = jnp.dot(t, w2_ref[...], preferred_element_type=jnp.float32)
    g2q_ref[...] = (h2 * dinv_ref[...]).astype(jnp.bfloat16)


def _tc3_body(agg_ref, dinv_ref, b2_ref, z_ref):
    z_ref[...] = dinv_ref[...] * (agg_ref[0] + agg_ref[1]) + b2_ref[...]


def kernel(x, edge_index, W1, b1, W2, b2):
    n_edges = edge_index.shape[1]
    hidden = W1.shape[1]
    z_dim = W2.shape[1]
    n_tot = n_edges + N_NODES
    per_tile = -(-n_tot // (NW * CHUNK * NB)) * NB
    m0, m1 = _split(per_tile)
    n_rows = NS * (m0 + m1) + max(0, m0 - m1)
    e_pad = n_rows * CHUNK

    loop = jnp.arange(N_NODES, dtype=jnp.int32)
    pad_s = jnp.zeros((e_pad - n_tot,), jnp.int32)
    pad_d = jnp.full((e_pad - n_tot,), N_NODES, jnp.int32)
    src2 = jnp.concatenate([edge_index[0], loop, pad_s]).reshape(n_rows, CHUNK)
    dst2 = jnp.concatenate([edge_index[1], loop, pad_d]).reshape(n_rows, CHUNK)

    degp = _make_deg_call(m0, m1)(dst2)
    degt = degp.reshape(NW, NPAD)[:, :N_NODES].T

    W1b = W1[:, _interleave_cols(hidden)]
    W2b = W2[:, _interleave_cols(z_dim)]

    g1q, dinv = pl.pallas_call(
        _tc1_body,
        out_shape=(
            jax.ShapeDtypeStruct((N_NODES, hidden), jnp.bfloat16),
            jax.ShapeDtypeStruct((N_NODES, 1), jnp.float32),
        ),
    )(x, W1b, degt)

    agg1 = _make_agg_call(m0, m1, hidden)(g1q, src2, dst2)
    agg1 = agg1.reshape(NC, NPAD, hidden)[:, :N_NODES]

    g2q = pl.pallas_call(
        _tc2_body,
        out_shape=jax.ShapeDtypeStruct((N_NODES, z_dim), jnp.bfloat16),
    )(agg1, dinv, b1.reshape(1, hidden), W2b)

    agg2 = _make_agg_call(m0, m1, z_dim)(g2q, src2, dst2)
    agg2 = agg2.reshape(NC, NPAD, z_dim)[:, :N_NODES]

    z = pl.pallas_call(
        _tc3_body,
        out_shape=jax.ShapeDtypeStruct((N_NODES, z_dim), jnp.float32),
    )(agg2, dinv, b2.reshape(1, z_dim))

    return z


# speedup vs baseline: 1.1036x; 1.0209x over previous
import functools

import numpy as np
import jax
import jax.numpy as jnp
from jax import lax
from jax.experimental import pallas as pl
from jax.experimental.pallas import tpu as pltpu
from jax.experimental.pallas import tpu_sc as plsc

N_NODES = 10000
NC, NS, L = 2, 16, 16
NW = NC * NS
CHUNK = 128
NB = 4
ROWS_PER_TILE = 632
NPAD = NS * ROWS_PER_TILE

_MESH = plsc.VectorSubcoreMesh(
    core_axis_name="c", subcore_axis_name="s", num_cores=NC, num_subcores=NS
)
_SC_PARAMS = pltpu.CompilerParams(
    needs_layout_passes=False, use_tc_tiling_on_sc=False
)

_CORE0_FRAC = 0.5


def _split(per_tile):
    total = 2 * per_tile
    m0 = int(round(total * _CORE0_FRAC / float(NB))) * NB
    m0 = min(max(m0, NB), total - NB)
    return m0, total - m0


def _interleave_cols(feat):
    border = []
    for g in range(feat // 32):
        for j in range(16):
            border.append(32 * g + j)
            border.append(32 * g + 16 + j)
    return np.asarray(border, dtype=np.int32)


def _tile_coords():
    c_ax = lax.axis_index("c")
    s_ax = lax.axis_index("s")
    return c_ax, s_ax


def _deg_body(m0, m1, dst2, degp, deg_v, didx2, si):
    c_ax, s_ax = _tile_coords()
    wid = c_ax * NS + s_ax
    mm = max(m0, m1)
    is0 = c_ax == 0
    m = jnp.where(is0, m0, m1)
    base = jnp.where(is0, s_ax * m0, NS * m0 + s_ax * m1)

    cp = pltpu.async_copy(dst2.at[pl.ds(pl.multiple_of(base, NB), mm)], didx2, si)

    def zero(r, carry):
        deg_v[pl.ds(r * L, L)] = jnp.zeros((L,), jnp.float32)
        return carry

    lax.fori_loop(0, NPAD // L, zero, 0)
    cp.wait()

    ones = jnp.ones((L,), jnp.float32)

    def body(c, carry):
        for k in range(CHUNK // L):
            idx = didx2[c, pl.ds(k * L, L)]
            plsc.addupdate_scatter(deg_v, [idx], ones)
        return carry

    lax.fori_loop(0, m, body, 0)
    out_off = pl.multiple_of(wid * NPAD, 8)
    pltpu.sync_copy(deg_v, degp.at[pl.ds(out_off, NPAD)])


def _make_deg_call(m0, m1):
    return pl.kernel(
        functools.partial(_deg_body, m0, m1),
        out_type=jax.ShapeDtypeStruct((NW * NPAD,), jnp.float32),
        mesh=_MESH,
        scratch_types=[
            pltpu.VMEM((NPAD,), jnp.float32),
            pltpu.VMEM((max(m0, m1), CHUNK), jnp.int32),
            pltpu.SemaphoreType.DMA,
        ],
        compiler_params=_SC_PARAMS,
    )


def _agg_body(m0, m1, feat, g_hbm, src2, dst2, out_hbm,
              acc_sh, sidx2, didx2,
              rb0, rb1, rb2, rb3, rf0, rf1, rf2, rf3,
              si, sg0, sg1, sg2, sg3, ss0, ss1, ss2, ss3):
    rows_bf = (rb0, rb1, rb2, rb3)
    rows_f = (rf0, rf1, rf2, rf3)
    sg = (sg0, sg1, sg2, sg3)
    ss = (ss0, ss1, ss2, ss3)
    c_ax, s_ax = _tile_coords()
    mm = max(m0, m1)
    is0 = c_ax == 0
    m = jnp.where(is0, m0, m1)
    base = jnp.where(is0, s_ax * m0, NS * m0 + s_ax * m1)

    islab = pl.multiple_of(base, NB)
    cp_s = pltpu.async_copy(src2.at[pl.ds(islab, mm)], sidx2, si)
    cp_d = pltpu.async_copy(dst2.at[pl.ds(islab, mm)], didx2, si)

    def zero(r, carry):
        for k in range(feat // L):
            rf0[r, pl.ds(k * L, L)] = jnp.zeros((L,), jnp.float32)
        return carry

    lax.fori_loop(0, CHUNK, zero, 0)
    slab = pl.multiple_of(s_ax * ROWS_PER_TILE, 8)
    n_full, rem = ROWS_PER_TILE // CHUNK, ROWS_PER_TILE % CHUNK
    for q in range(n_full):
        pltpu.sync_copy(rf0, acc_sh.at[pl.ds(slab + q * CHUNK, CHUNK)])
    if rem:
        pltpu.sync_copy(
            rf0.at[pl.ds(0, rem)],
            acc_sh.at[pl.ds(slab + n_full * CHUNK, rem)],
        )
    cp_s.wait()
    cp_d.wait()
    plsc.subcore_barrier()

    hi_mask = jnp.full((L,), -65536, jnp.int32)

    def gather_start(c, b):
        pltpu.async_copy(g_hbm.at[sidx2.at[c]], rows_bf[b], sg[b])

    def gather_wait(b):
        pltpu.make_async_copy(g_hbm.at[sidx2.at[0]], rows_bf[b], sg[b]).wait()

    def expand(b):
        src = rows_bf[b]
        dstf = rows_f[b]

        def erow(r, carry):
            for g in range(feat // 32):
                v = plsc.bitcast(src[r, pl.ds(32 * g, 32)], jnp.int32)
                lo = plsc.bitcast(v << 16, jnp.float32)
                hi = plsc.bitcast(v & hi_mask, jnp.float32)
                dstf[r, pl.ds(32 * g, L)] = lo
                dstf[r, pl.ds(32 * g + L, L)] = hi
            return carry

        lax.fori_loop(0, CHUNK, erow, 0)

    def scatter_start(c, b):
        pltpu.async_copy(rows_f[b], acc_sh.at[didx2.at[c]], ss[b], add=True)

    def scatter_wait(b):
        pltpu.make_async_copy(rows_f[b], acc_sh.at[pl.ds(0, CHUNK)], ss[b]).wait()

    def outer(t, carry):
        for b in range(NB):
            c = t * NB + b
            pd = (b + 1) % NB

            @pl.when(t > 0)
            def _():
                scatter_wait(b)

            gather_start(c, b)
            if b == NB - 1:
                gather_wait(pd)
                expand(pd)
                scatter_start(c - (NB - 1), pd)
            else:
                @pl.when(t > 0)
                def _():
                    gather_wait(pd)
                    expand(pd)
                    scatter_start(c - (NB - 1), pd)
        return carry

    lax.fori_loop(0, m // NB, outer, 0)

    for k in range(NB - 1):
        b = (k + 1) % NB
        gather_wait(b)
        expand(b)
        scatter_start(m - (NB - 1) + k, b)
    for b in range(NB):
        scatter_wait(b)
    plsc.subcore_barrier()
    pltpu.sync_copy(acc_sh.at[pl.ds(slab, ROWS_PER_TILE)], out_hbm.at[c_ax, s_ax])


def _make_agg_call(m0, m1, feat):
    mm = max(m0, m1)
    return pl.kernel(
        functools.partial(_agg_body, m0, m1, feat),
        out_type=jax.ShapeDtypeStruct((NC, NS, ROWS_PER_TILE, feat), jnp.float32),
        mesh=_MESH,
        scratch_types=[
            pltpu.VMEM_SHARED((NPAD, feat), jnp.float32),
            pltpu.VMEM((mm, CHUNK), jnp.int32),
            pltpu.VMEM((mm, CHUNK), jnp.int32),
            pltpu.VMEM((CHUNK, feat), jnp.bfloat16),
            pltpu.VMEM((CHUNK, feat), jnp.bfloat16),
            pltpu.VMEM((CHUNK, feat), jnp.bfloat16),
            pltpu.VMEM((CHUNK, feat), jnp.bfloat16),
            pltpu.VMEM((CHUNK, feat), jnp.float32),
            pltpu.VMEM((CHUNK, feat), jnp.float32),
            pltpu.VMEM((CHUNK, feat), jnp.float32),
            pltpu.VMEM((CHUNK, feat), jnp.float32),
        ] + [pltpu.SemaphoreType.DMA] * 9,
        compiler_params=_SC_PARAMS,
    )


def _tc1_body(x_ref, w1_ref, degt_ref, g1q_ref, dinv_ref):
    deg = jnp.sum(degt_ref[...], axis=1, keepdims=True)
    dinv = lax.rsqrt(jnp.maximum(deg, 1e-12))
    h = jnp.dot(x_ref[...], w1_ref[...], preferred_element_type=jnp.float32)
    g1q_ref[...] = (h * dinv).astype(jnp.bfloat16)
    dinv_ref[...] = dinv


def _tc2_body(agg_ref, dinv_ref, b1_ref, w2_ref, g2q_ref):
    y = agg_ref[0] + agg_ref[1]
    t = jnp.maximum(dinv_ref[...] * y + b1_ref[...], 0.0)
    h2 = jnp.dot(t, w2_ref[...], preferred_element_type=jnp.float32)
    g2q_ref[...] = (h2 * dinv_ref[...]).astype(jnp.bfloat16)


def _tc3_body(agg_ref, dinv_ref, b2_ref, z_ref):
    z_ref[...] = dinv_ref[...] * (agg_ref[0] + agg_ref[1]) + b2_ref[...]


def kernel(x, edge_index, W1, b1, W2, b2):
    n_edges = edge_index.shape[1]
    hidden = W1.shape[1]
    z_dim = W2.shape[1]
    n_tot = n_edges + N_NODES
    per_tile = -(-n_tot // (NW * CHUNK * NB)) * NB
    m0, m1 = _split(per_tile)
    n_rows = NS * (m0 + m1) + max(0, m0 - m1)
    e_pad = n_rows * CHUNK

    loop = jnp.arange(N_NODES, dtype=jnp.int32)
    pad_s = jnp.zeros((e_pad - n_tot,), jnp.int32)
    pad_d = jnp.full((e_pad - n_tot,), N_NODES, jnp.int32)
    src2 = jnp.concatenate([edge_index[0], loop, pad_s]).reshape(n_rows, CHUNK)
    dst2 = jnp.concatenate([edge_index[1], loop, pad_d]).reshape(n_rows, CHUNK)

    degp = _make_deg_call(m0, m1)(dst2)
    degt = degp.reshape(NW, NPAD)[:, :N_NODES].T

    W1b = W1[:, _interleave_cols(hidden)]
    W2b = W2[:, _interleave_cols(z_dim)]

    g1q, dinv = pl.pallas_call(
        _tc1_body,
        out_shape=(
            jax.ShapeDtypeStruct((N_NODES, hidden), jnp.bfloat16),
            jax.ShapeDtypeStruct((N_NODES, 1), jnp.float32),
        ),
    )(x, W1b, degt)

    agg1 = _make_agg_call(m0, m1, hidden)(g1q, src2, dst2)
    agg1 = agg1.reshape(NC, NPAD, hidden)[:, :N_NODES]

    g2q = pl.pallas_call(
        _tc2_body,
        out_shape=jax.ShapeDtypeStruct((N_NODES, z_dim), jnp.bfloat16),
    )(agg1, dinv, b1.reshape(1, hidden), W2b)

    agg2 = _make_agg_call(m0, m1, z_dim)(g2q, src2, dst2)
    agg2 = agg2.reshape(NC, NPAD, z_dim)[:, :N_NODES]

    z = pl.pallas_call(
        _tc3_body,
        out_shape=jax.ShapeDtypeStruct((N_NODES, z_dim), jnp.float32),
    )(agg2, dinv, b2.reshape(1, z_dim))

    return z


# speedup vs baseline: 1.3406x; 1.2148x over previous
import functools

import numpy as np
import jax
import jax.numpy as jnp
from jax import lax
from jax.experimental import pallas as pl
from jax.experimental.pallas import tpu as pltpu
from jax.experimental.pallas import tpu_sc as plsc

N_NODES = 10000
NC, NS, L = 2, 16, 16
NW = NC * NS
CHUNK = 128
NB = 4
ROWS_PER_TILE = 626
NPAD = NS * ROWS_PER_TILE
STAGE = N_NODES // NS

_MESH = plsc.VectorSubcoreMesh(
    core_axis_name="c", subcore_axis_name="s", num_cores=NC, num_subcores=NS
)
_SC_PARAMS = pltpu.CompilerParams(
    needs_layout_passes=False, use_tc_tiling_on_sc=False
)

_CORE0_FRAC = 0.5


def _split(per_tile):
    total = 2 * per_tile
    m0 = int(round(total * _CORE0_FRAC / float(NB))) * NB
    m0 = min(max(m0, NB), total - NB)
    return m0, total - m0


def _interleave_cols(feat):
    border = []
    for g in range(feat // 32):
        for j in range(16):
            border.append(32 * g + j)
            border.append(32 * g + 16 + j)
    return np.asarray(border, dtype=np.int32)


def _tile_coords():
    c_ax = lax.axis_index("c")
    s_ax = lax.axis_index("s")
    return c_ax, s_ax


def _deg_body(m0, m1, dst2, degp, deg_v, didx2, si):
    c_ax, s_ax = _tile_coords()
    wid = c_ax * NS + s_ax
    mm = max(m0, m1)
    is0 = c_ax == 0
    m = jnp.where(is0, m0, m1)
    base = jnp.where(is0, s_ax * m0, NS * m0 + s_ax * m1)

    cp = pltpu.async_copy(dst2.at[pl.ds(pl.multiple_of(base, NB), mm)], didx2, si)

    def zero(r, carry):
        deg_v[pl.ds(r * L, L)] = jnp.zeros((L,), jnp.float32)
        return carry

    lax.fori_loop(0, NPAD // L, zero, 0)
    cp.wait()

    ones = jnp.ones((L,), jnp.float32)

    def body(c, carry):
        for k in range(CHUNK // L):
            idx = didx2[c, pl.ds(k * L, L)]
            plsc.addupdate_scatter(deg_v, [idx], ones)
        return carry

    lax.fori_loop(0, m, body, 0)
    out_off = pl.multiple_of(wid * NPAD, 8)
    pltpu.sync_copy(deg_v, degp.at[pl.ds(out_off, NPAD)])


def _make_deg_call(m0, m1):
    return pl.kernel(
        functools.partial(_deg_body, m0, m1),
        out_type=jax.ShapeDtypeStruct((NW * NPAD,), jnp.float32),
        mesh=_MESH,
        scratch_types=[
            pltpu.VMEM((NPAD,), jnp.float32),
            pltpu.VMEM((max(m0, m1), CHUNK), jnp.int32),
            pltpu.SemaphoreType.DMA,
        ],
        compiler_params=_SC_PARAMS,
    )


def _agg_body(m0, m1, feat, g_hbm, src2, dst2, out_hbm,
              acc_sh, g_sh, sidx2, didx2,
              rb0, rb1, rb2, rb3, rf0, rf1, rf2, rf3,
              si, st, sg0, sg1, sg2, sg3, ss0, ss1, ss2, ss3):
    rows_bf = (rb0, rb1, rb2, rb3)
    rows_f = (rf0, rf1, rf2, rf3)
    sg = (sg0, sg1, sg2, sg3)
    ss = (ss0, ss1, ss2, ss3)
    c_ax, s_ax = _tile_coords()
    mm = max(m0, m1)
    is0 = c_ax == 0
    m = jnp.where(is0, m0, m1)
    base = jnp.where(is0, s_ax * m0, NS * m0 + s_ax * m1)

    islab = pl.multiple_of(base, NB)
    cp_s = pltpu.async_copy(src2.at[pl.ds(islab, mm)], sidx2, si)
    cp_d = pltpu.async_copy(dst2.at[pl.ds(islab, mm)], didx2, si)
    cp_t = pltpu.async_copy(
        g_hbm.at[pl.ds(s_ax * STAGE, STAGE)],
        g_sh.at[pl.ds(s_ax * STAGE, STAGE)],
        st,
    )

    def zero(r, carry):
        for k in range(feat // L):
            rf0[r, pl.ds(k * L, L)] = jnp.zeros((L,), jnp.float32)
        return carry

    lax.fori_loop(0, CHUNK, zero, 0)
    slab = pl.multiple_of(s_ax * ROWS_PER_TILE, 8)
    n_full, rem = ROWS_PER_TILE // CHUNK, ROWS_PER_TILE % CHUNK
    for q in range(n_full):
        pltpu.sync_copy(rf0, acc_sh.at[pl.ds(slab + q * CHUNK, CHUNK)])
    if rem:
        pltpu.sync_copy(
            rf0.at[pl.ds(0, rem)],
            acc_sh.at[pl.ds(slab + n_full * CHUNK, rem)],
        )
    cp_s.wait()
    cp_d.wait()
    cp_t.wait()
    plsc.subcore_barrier()

    hi_mask = jnp.full((L,), -65536, jnp.int32)

    def gather_start(c, b):
        pltpu.async_copy(g_sh.at[sidx2.at[c]], rows_bf[b], sg[b])

    def gather_wait(b):
        pltpu.make_async_copy(g_sh.at[sidx2.at[0]], rows_bf[b], sg[b]).wait()

    def expand(b):
        src = rows_bf[b]
        dstf = rows_f[b]

        def erow(r, carry):
            for g in range(feat // 32):
                v = plsc.bitcast(src[r, pl.ds(32 * g, 32)], jnp.int32)
                lo = plsc.bitcast(v << 16, jnp.float32)
                hi = plsc.bitcast(v & hi_mask, jnp.float32)
                dstf[r, pl.ds(32 * g, L)] = lo
                dstf[r, pl.ds(32 * g + L, L)] = hi
            return carry

        lax.fori_loop(0, CHUNK, erow, 0)

    def scatter_start(c, b):
        pltpu.async_copy(rows_f[b], acc_sh.at[didx2.at[c]], ss[b], add=True)

    def scatter_wait(b):
        pltpu.make_async_copy(rows_f[b], acc_sh.at[pl.ds(0, CHUNK)], ss[b]).wait()

    def outer(t, carry):
        for b in range(NB):
            c = t * NB + b
            pd = (b + 1) % NB

            @pl.when(t > 0)
            def _():
                scatter_wait(b)

            gather_start(c, b)
            if b == NB - 1:
                gather_wait(pd)
                expand(pd)
                scatter_start(c - (NB - 1), pd)
            else:
                @pl.when(t > 0)
                def _():
                    gather_wait(pd)
                    expand(pd)
                    scatter_start(c - (NB - 1), pd)
        return carry

    lax.fori_loop(0, m // NB, outer, 0)

    for k in range(NB - 1):
        b = (k + 1) % NB
        gather_wait(b)
        expand(b)
        scatter_start(m - (NB - 1) + k, b)
    for b in range(NB):
        scatter_wait(b)
    plsc.subcore_barrier()
    pltpu.sync_copy(acc_sh.at[pl.ds(slab, ROWS_PER_TILE)], out_hbm.at[c_ax, s_ax])


def _make_agg_call(m0, m1, feat):
    mm = max(m0, m1)
    return pl.kernel(
        functools.partial(_agg_body, m0, m1, feat),
        out_type=jax.ShapeDtypeStruct((NC, NS, ROWS_PER_TILE, feat), jnp.float32),
        mesh=_MESH,
        scratch_types=[
            pltpu.VMEM_SHARED((NPAD, feat), jnp.float32),
            pltpu.VMEM_SHARED((N_NODES, feat), jnp.bfloat16),
            pltpu.VMEM((mm, CHUNK), jnp.int32),
            pltpu.VMEM((mm, CHUNK), jnp.int32),
            pltpu.VMEM((CHUNK, feat), jnp.bfloat16),
            pltpu.VMEM((CHUNK, feat), jnp.bfloat16),
            pltpu.VMEM((CHUNK, feat), jnp.bfloat16),
            pltpu.VMEM((CHUNK, feat), jnp.bfloat16),
            pltpu.VMEM((CHUNK, feat), jnp.float32),
            pltpu.VMEM((CHUNK, feat), jnp.float32),
            pltpu.VMEM((CHUNK, feat), jnp.float32),
            pltpu.VMEM((CHUNK, feat), jnp.float32),
        ] + [pltpu.SemaphoreType.DMA] * 10,
        compiler_params=_SC_PARAMS,
    )


def _tc1_body(x_ref, w1_ref, degt_ref, g1q_ref, dinv_ref):
    deg = jnp.sum(degt_ref[...], axis=1, keepdims=True)
    dinv = lax.rsqrt(jnp.maximum(deg, 1e-12))
    h = jnp.dot(x_ref[...], w1_ref[...], preferred_element_type=jnp.float32)
    g1q_ref[...] = (h * dinv).astype(jnp.bfloat16)
    dinv_ref[...] = dinv


def _tc2_body(agg_ref, dinv_ref, b1_ref, w2_ref, g2q_ref):
    y = agg_ref[0] + agg_ref[1]
    t = jnp.maximum(dinv_ref[...] * y + b1_ref[...], 0.0)
    h2 = jnp.dot(t, w2_ref[...], preferred_element_type=jnp.float32)
    g2q_ref[...] = (h2 * dinv_ref[...]).astype(jnp.bfloat16)


def _tc3_body(agg_ref, dinv_ref, b2_ref, z_ref):
    z_ref[...] = dinv_ref[...] * (agg_ref[0] + agg_ref[1]) + b2_ref[...]


def kernel(x, edge_index, W1, b1, W2, b2):
    n_edges = edge_index.shape[1]
    hidden = W1.shape[1]
    z_dim = W2.shape[1]
    n_tot = n_edges + N_NODES
    per_tile = -(-n_tot // (NW * CHUNK * NB)) * NB
    m0, m1 = _split(per_tile)
    n_rows = NS * (m0 + m1) + max(0, m0 - m1)
    e_pad = n_rows * CHUNK

    loop = jnp.arange(N_NODES, dtype=jnp.int32)
    pad_s = jnp.zeros((e_pad - n_tot,), jnp.int32)
    pad_d = jnp.full((e_pad - n_tot,), N_NODES, jnp.int32)
    src2 = jnp.concatenate([edge_index[0], loop, pad_s]).reshape(n_rows, CHUNK)
    dst2 = jnp.concatenate([edge_index[1], loop, pad_d]).reshape(n_rows, CHUNK)

    degp = _make_deg_call(m0, m1)(dst2)
    degt = degp.reshape(NW, NPAD)[:, :N_NODES].T

    W1b = W1[:, _interleave_cols(hidden)]
    W2b = W2[:, _interleave_cols(z_dim)]

    g1q, dinv = pl.pallas_call(
        _tc1_body,
        out_shape=(
            jax.ShapeDtypeStruct((N_NODES, hidden), jnp.bfloat16),
            jax.ShapeDtypeStruct((N_NODES, 1), jnp.float32),
        ),
    )(x, W1b, degt)

    agg1 = _make_agg_call(m0, m1, hidden)(g1q, src2, dst2)
    agg1 = agg1.reshape(NC, NPAD, hidden)[:, :N_NODES]

    g2q = pl.pallas_call(
        _tc2_body,
        out_shape=jax.ShapeDtypeStruct((N_NODES, z_dim), jnp.bfloat16),
    )(agg1, dinv, b1.reshape(1, hidden), W2b)

    agg2 = _make_agg_call(m0, m1, z_dim)(g2q, src2, dst2)
    agg2 = agg2.reshape(NC, NPAD, z_dim)[:, :N_NODES]

    z = pl.pallas_call(
        _tc3_body,
        out_shape=jax.ShapeDtypeStruct((N_NODES, z_dim), jnp.float32),
    )(agg2, dinv, b2.reshape(1, z_dim))

    return z


# speedup vs baseline: 1.9244x; 1.4355x over previous
import functools

import jax
import jax.numpy as jnp
from jax import lax
from jax.experimental import pallas as pl
from jax.experimental.pallas import tpu as pltpu
from jax.experimental.pallas import tpu_sc as plsc

N_NODES = 10000
NC, NS, L = 2, 16, 16
NW = NC * NS
CHUNK = 128
NB = 3
ROWS_PER_TILE = 626
NPAD = NS * ROWS_PER_TILE
STAGE = N_NODES // NS

_MESH = plsc.VectorSubcoreMesh(
    core_axis_name="c", subcore_axis_name="s", num_cores=NC, num_subcores=NS
)
_SC_PARAMS = pltpu.CompilerParams(
    needs_layout_passes=False, use_tc_tiling_on_sc=False
)

_CORE0_FRAC = 0.5


def _split(per_tile):
    total = 2 * per_tile
    m0 = int(round(total * _CORE0_FRAC / float(NB))) * NB
    m0 = min(max(m0, NB), total - NB)
    return m0, total - m0


def _tile_coords():
    c_ax = lax.axis_index("c")
    s_ax = lax.axis_index("s")
    return c_ax, s_ax


def _deg_body(m0, m1, dst2, degp, deg_v, didx2, si):
    c_ax, s_ax = _tile_coords()
    wid = c_ax * NS + s_ax
    mm = max(m0, m1)
    is0 = c_ax == 0
    m = jnp.where(is0, m0, m1)
    base = jnp.where(is0, s_ax * m0, NS * m0 + s_ax * m1)

    cp = pltpu.async_copy(dst2.at[pl.ds(pl.multiple_of(base, NB), mm)], didx2, si)

    def zero(r, carry):
        deg_v[pl.ds(r * L, L)] = jnp.zeros((L,), jnp.float32)
        return carry

    lax.fori_loop(0, NPAD // L, zero, 0)
    cp.wait()

    ones = jnp.ones((L,), jnp.float32)

    def body(c, carry):
        for k in range(CHUNK // L):
            idx = didx2[c, pl.ds(k * L, L)]
            plsc.addupdate_scatter(deg_v, [idx], ones)
        return carry

    lax.fori_loop(0, m, body, 0)
    out_off = pl.multiple_of(wid * NPAD, 8)
    pltpu.sync_copy(deg_v, degp.at[pl.ds(out_off, NPAD)])


def _make_deg_call(m0, m1):
    return pl.kernel(
        functools.partial(_deg_body, m0, m1),
        out_type=jax.ShapeDtypeStruct((NW * NPAD,), jnp.float32),
        mesh=_MESH,
        scratch_types=[
            pltpu.VMEM((NPAD,), jnp.float32),
            pltpu.VMEM((max(m0, m1), CHUNK), jnp.int32),
            pltpu.SemaphoreType.DMA,
        ],
        compiler_params=_SC_PARAMS,
    )


def _agg_body(m0, m1, feat, g_hbm, src2, dst2, out_hbm,
              acc_sh, g_sh, sidx2, didx2, rf0, rf1, rf2,
              si, st, sg0, sg1, sg2, ss0, ss1, ss2):
    rows_f = (rf0, rf1, rf2)
    sg = (sg0, sg1, sg2)
    ss = (ss0, ss1, ss2)
    c_ax, s_ax = _tile_coords()
    mm = max(m0, m1)
    is0 = c_ax == 0
    m = jnp.where(is0, m0, m1)
    base = jnp.where(is0, s_ax * m0, NS * m0 + s_ax * m1)

    islab = pl.multiple_of(base, NB)
    cp_s = pltpu.async_copy(src2.at[pl.ds(islab, mm)], sidx2, si)
    cp_d = pltpu.async_copy(dst2.at[pl.ds(islab, mm)], didx2, si)
    cp_t = pltpu.async_copy(
        g_hbm.at[pl.ds(s_ax * STAGE, STAGE)],
        g_sh.at[pl.ds(s_ax * STAGE, STAGE)],
        st,
    )

    def zero(r, carry):
        for k in range(feat // L):
            rf0[r, pl.ds(k * L, L)] = jnp.zeros((L,), jnp.float32)
        return carry

    lax.fori_loop(0, CHUNK, zero, 0)
    slab = pl.multiple_of(s_ax * ROWS_PER_TILE, 2)
    n_full, rem = ROWS_PER_TILE // CHUNK, ROWS_PER_TILE % CHUNK
    for q in range(n_full):
        pltpu.sync_copy(rf0, acc_sh.at[pl.ds(slab + q * CHUNK, CHUNK)])
    if rem:
        pltpu.sync_copy(
            rf0.at[pl.ds(0, rem)],
            acc_sh.at[pl.ds(slab + n_full * CHUNK, rem)],
        )
    cp_s.wait()
    cp_d.wait()
    cp_t.wait()
    plsc.subcore_barrier()

    def gather_start(c, b):
        pltpu.async_copy(g_sh.at[sidx2.at[c]], rows_f[b], sg[b])

    def gather_wait(b):
        pltpu.make_async_copy(g_sh.at[sidx2.at[0]], rows_f[b], sg[b]).wait()

    def scatter_start(c, b):
        pltpu.async_copy(rows_f[b], acc_sh.at[didx2.at[c]], ss[b], add=True)

    def scatter_wait(b):
        pltpu.make_async_copy(rows_f[b], acc_sh.at[pl.ds(0, CHUNK)], ss[b]).wait()

    def outer(t, carry):
        for b in range(NB):
            c = t * NB + b
            pd = (b + 1) % NB

            @pl.when(t > 0)
            def _():
                scatter_wait(b)

            gather_start(c, b)
            if b == NB - 1:
                gather_wait(pd)
                scatter_start(c - (NB - 1), pd)
            else:
                @pl.when(t > 0)
                def _():
                    gather_wait(pd)
                    scatter_start(c - (NB - 1), pd)
        return carry

    lax.fori_loop(0, m // NB, outer, 0)

    for k in range(NB - 1):
        b = (k + 1) % NB
        gather_wait(b)
        scatter_start(m - (NB - 1) + k, b)
    for b in range(NB):
        scatter_wait(b)
    plsc.subcore_barrier()
    pltpu.sync_copy(acc_sh.at[pl.ds(slab, ROWS_PER_TILE)], out_hbm.at[c_ax, s_ax])


def _make_agg_call(m0, m1, feat):
    mm = max(m0, m1)
    return pl.kernel(
        functools.partial(_agg_body, m0, m1, feat),
        out_type=jax.ShapeDtypeStruct((NC, NS, ROWS_PER_TILE, feat), jnp.float32),
        mesh=_MESH,
        scratch_types=[
            pltpu.VMEM_SHARED((NPAD, feat), jnp.float32),
            pltpu.VMEM_SHARED((N_NODES, feat), jnp.float32),
            pltpu.VMEM((mm, CHUNK), jnp.int32),
            pltpu.VMEM((mm, CHUNK), jnp.int32),
            pltpu.VMEM((CHUNK, feat), jnp.float32),
            pltpu.VMEM((CHUNK, feat), jnp.float32),
            pltpu.VMEM((CHUNK, feat), jnp.float32),
        ] + [pltpu.SemaphoreType.DMA] * 8,
        compiler_params=_SC_PARAMS,
    )


def _tc1_body(x_ref, w1_ref, degt_ref, g1_ref, dinv_ref):
    deg = jnp.sum(degt_ref[...], axis=1, keepdims=True)
    dinv = lax.rsqrt(jnp.maximum(deg, 1e-12))
    h = jnp.dot(x_ref[...], w1_ref[...], preferred_element_type=jnp.float32)
    g1_ref[...] = h * dinv
    dinv_ref[...] = dinv


def _tc2_body(agg_ref, dinv_ref, b1_ref, w2_ref, g2_ref):
    y = agg_ref[0] + agg_ref[1]
    t = jnp.maximum(dinv_ref[...] * y + b1_ref[...], 0.0)
    h2 = jnp.dot(t, w2_ref[...], preferred_element_type=jnp.float32)
    g2_ref[...] = h2 * dinv_ref[...]


def _tc3_body(agg_ref, dinv_ref, b2_ref, z_ref):
    z_ref[...] = dinv_ref[...] * (agg_ref[0] + agg_ref[1]) + b2_ref[...]


def kernel(x, edge_index, W1, b1, W2, b2):
    n_edges = edge_index.shape[1]
    hidden = W1.shape[1]
    z_dim = W2.shape[1]
    n_tot = n_edges + N_NODES
    per_tile = -(-n_tot // (NW * CHUNK * NB)) * NB
    m0, m1 = _split(per_tile)
    n_rows = NS * (m0 + m1) + max(0, m0 - m1)
    e_pad = n_rows * CHUNK

    loop = jnp.arange(N_NODES, dtype=jnp.int32)
    pad_s = jnp.zeros((e_pad - n_tot,), jnp.int32)
    pad_d = jnp.full((e_pad - n_tot,), N_NODES, jnp.int32)
    src2 = jnp.concatenate([edge_index[0], loop, pad_s]).reshape(n_rows, CHUNK)
    dst2 = jnp.concatenate([edge_index[1], loop, pad_d]).reshape(n_rows, CHUNK)

    degp = _make_deg_call(m0, m1)(dst2)
    degt = degp.reshape(NW, NPAD)[:, :N_NODES].T

    g1, dinv = pl.pallas_call(
        _tc1_body,
        out_shape=(
            jax.ShapeDtypeStruct((N_NODES, hidden), jnp.float32),
            jax.ShapeDtypeStruct((N_NODES, 1), jnp.float32),
        ),
    )(x, W1, degt)

    agg1 = _make_agg_call(m0, m1, hidden)(g1, src2, dst2)
    agg1 = agg1.reshape(NC, NPAD, hidden)[:, :N_NODES]

    g2 = pl.pallas_call(
        _tc2_body,
        out_shape=jax.ShapeDtypeStruct((N_NODES, z_dim), jnp.float32),
    )(agg1, dinv, b1.reshape(1, hidden), W2)

    agg2 = _make_agg_call(m0, m1, z_dim)(g2, src2, dst2)
    agg2 = agg2.reshape(NC, NPAD, z_dim)[:, :N_NODES]

    z = pl.pallas_call(
        _tc3_body,
        out_shape=jax.ShapeDtypeStruct((N_NODES, z_dim), jnp.float32),
    )(agg2, dinv, b2.reshape(1, z_dim))

    return z


# speedup vs baseline: 2.0830x; 1.0824x over previous
import functools

import jax
import jax.numpy as jnp
from jax import lax
from jax.experimental import pallas as pl
from jax.experimental.pallas import tpu as pltpu
from jax.experimental.pallas import tpu_sc as plsc

N_NODES = 10000
NC, NS, L = 2, 16, 16
NW = NC * NS
CHUNK = 128
NB = 3
ROWS_PER_TILE = 632
NPAD = NS * ROWS_PER_TILE
STAGE = N_NODES // NS

_MESH = plsc.VectorSubcoreMesh(
    core_axis_name="c", subcore_axis_name="s", num_cores=NC, num_subcores=NS
)
_SC_PARAMS = pltpu.CompilerParams(
    needs_layout_passes=False, use_tc_tiling_on_sc=False
)

_CORE0_FRAC = 0.5


def _split(per_tile):
    total = 2 * per_tile
    m0 = int(round(total * _CORE0_FRAC / float(NB))) * NB
    m0 = min(max(m0, NB), total - NB)
    return m0, total - m0


def _tile_coords():
    c_ax = lax.axis_index("c")
    s_ax = lax.axis_index("s")
    return c_ax, s_ax


def _deg_body(m0, m1, dst2, degp, deg_v, didx2, si):
    c_ax, s_ax = _tile_coords()
    wid = c_ax * NS + s_ax
    mm = max(m0, m1)
    is0 = c_ax == 0
    m = jnp.where(is0, m0, m1)
    base = jnp.where(is0, s_ax * m0, NS * m0 + s_ax * m1)

    cp = pltpu.async_copy(dst2.at[pl.ds(pl.multiple_of(base, NB), mm)], didx2, si)

    def zero(r, carry):
        deg_v[pl.ds(r * L, L)] = jnp.zeros((L,), jnp.float32)
        return carry

    lax.fori_loop(0, NPAD // L, zero, 0)
    cp.wait()

    ones = jnp.ones((L,), jnp.float32)

    def body(c, carry):
        for k in range(CHUNK // L):
            idx = didx2[c, pl.ds(k * L, L)]
            plsc.addupdate_scatter(deg_v, [idx], ones)
        return carry

    lax.fori_loop(0, m, body, 0)
    out_off = pl.multiple_of(wid * NPAD, 8)
    pltpu.sync_copy(deg_v, degp.at[pl.ds(out_off, NPAD)])


def _make_deg_call(m0, m1):
    return pl.kernel(
        functools.partial(_deg_body, m0, m1),
        out_type=jax.ShapeDtypeStruct((NW * NPAD,), jnp.float32),
        mesh=_MESH,
        scratch_types=[
            pltpu.VMEM((NPAD,), jnp.float32),
            pltpu.VMEM((max(m0, m1), CHUNK), jnp.int32),
            pltpu.SemaphoreType.DMA,
        ],
        compiler_params=_SC_PARAMS,
    )


def _agg_body(m0, m1, feat, g_hbm, src2, dst2, out_hbm,
              acc_sh, g_sh, sidx2, didx2, rf0, rf1, rf2,
              si, st, sg0, sg1, sg2, ss0, ss1, ss2):
    rows_f = (rf0, rf1, rf2)
    sg = (sg0, sg1, sg2)
    ss = (ss0, ss1, ss2)
    c_ax, s_ax = _tile_coords()
    mm = max(m0, m1)
    is0 = c_ax == 0
    m = jnp.where(is0, m0, m1)
    base = jnp.where(is0, s_ax * m0, NS * m0 + s_ax * m1)

    islab = pl.multiple_of(base, NB)
    cp_s = pltpu.async_copy(src2.at[pl.ds(islab, mm)], sidx2, si)
    cp_d = pltpu.async_copy(dst2.at[pl.ds(islab, mm)], didx2, si)
    cp_t = pltpu.async_copy(
        g_hbm.at[pl.ds(s_ax * STAGE, STAGE)],
        g_sh.at[pl.ds(s_ax * STAGE, STAGE)],
        st,
    )

    def zero(r, carry):
        for k in range(feat // L):
            rf0[r, pl.ds(k * L, L)] = jnp.zeros((L,), jnp.float32)
        return carry

    lax.fori_loop(0, CHUNK, zero, 0)
    slab = pl.multiple_of(s_ax * ROWS_PER_TILE, 8)
    n_full, rem = ROWS_PER_TILE // CHUNK, ROWS_PER_TILE % CHUNK
    for q in range(n_full):
        pltpu.sync_copy(rf0, acc_sh.at[pl.ds(slab + q * CHUNK, CHUNK)])
    if rem:
        pltpu.sync_copy(
            rf0.at[pl.ds(0, rem)],
            acc_sh.at[pl.ds(slab + n_full * CHUNK, rem)],
        )
    cp_s.wait()
    cp_d.wait()
    cp_t.wait()
    plsc.subcore_barrier()

    def gather_start(c, b):
        pltpu.async_copy(g_sh.at[sidx2.at[c]], rows_f[b], sg[b])

    def gather_wait(b):
        pltpu.make_async_copy(g_sh.at[sidx2.at[0]], rows_f[b], sg[b]).wait()

    def scatter_start(c, b):
        pltpu.async_copy(rows_f[b], acc_sh.at[didx2.at[c]], ss[b], add=True)

    def scatter_wait(b):
        pltpu.make_async_copy(rows_f[b], acc_sh.at[pl.ds(0, CHUNK)], ss[b]).wait()

    def outer(t, carry):
        for b in range(NB):
            c = t * NB + b
            pd = (b + 1) % NB

            @pl.when(t > 0)
            def _():
                scatter_wait(b)

            gather_start(c, b)
            if b == NB - 1:
                gather_wait(pd)
                scatter_start(c - (NB - 1), pd)
            else:
                @pl.when(t > 0)
                def _():
                    gather_wait(pd)
                    scatter_start(c - (NB - 1), pd)
        return carry

    lax.fori_loop(0, m // NB, outer, 0)

    for k in range(NB - 1):
        b = (k + 1) % NB
        gather_wait(b)
        scatter_start(m - (NB - 1) + k, b)
    for b in range(NB):
        scatter_wait(b)
    plsc.subcore_barrier()
    pltpu.sync_copy(acc_sh.at[pl.ds(slab, ROWS_PER_TILE)], out_hbm.at[c_ax, s_ax])


def _make_agg_call(m0, m1, feat):
    mm = max(m0, m1)
    return pl.kernel(
        functools.partial(_agg_body, m0, m1, feat),
        out_type=jax.ShapeDtypeStruct((NC, NS, ROWS_PER_TILE, feat), jnp.float32),
        mesh=_MESH,
        scratch_types=[
            pltpu.VMEM_SHARED((NPAD, feat), jnp.float32),
            pltpu.VMEM_SHARED((N_NODES, feat), jnp.float32),
            pltpu.VMEM((mm, CHUNK), jnp.int32),
            pltpu.VMEM((mm, CHUNK), jnp.int32),
            pltpu.VMEM((CHUNK, feat), jnp.float32),
            pltpu.VMEM((CHUNK, feat), jnp.float32),
            pltpu.VMEM((CHUNK, feat), jnp.float32),
        ] + [pltpu.SemaphoreType.DMA] * 8,
        compiler_params=_SC_PARAMS,
    )


def _dinv_col(degp):
    ones = jnp.ones((NW, 1), jnp.float32)
    deg = lax.dot_general(
        degp, ones, (((0,), (0,)), ((), ())),
        preferred_element_type=jnp.float32,
    )
    return lax.rsqrt(jnp.maximum(deg[:N_NODES], 1e-12))


def _tc1_body(x_ref, w1_ref, degp_ref, g1_ref):
    dinv = _dinv_col(degp_ref[...])
    h = jnp.dot(x_ref[...], w1_ref[...], preferred_element_type=jnp.float32)
    g1_ref[...] = h * dinv


def _tc2_body(agg_ref, degp_ref, b1_ref, w2_ref, g2_ref):
    dinv = _dinv_col(degp_ref[...])
    y = agg_ref[0, :N_NODES] + agg_ref[1, :N_NODES]
    t = jnp.maximum(dinv * y + b1_ref[...], 0.0)
    h2 = jnp.dot(t, w2_ref[...], preferred_element_type=jnp.float32)
    g2_ref[...] = h2 * dinv


def _tc3_body(agg_ref, degp_ref, b2_ref, z_ref):
    dinv = _dinv_col(degp_ref[...])
    z_ref[...] = dinv * (agg_ref[0, :N_NODES] + agg_ref[1, :N_NODES]) + b2_ref[...]


def kernel(x, edge_index, W1, b1, W2, b2):
    n_edges = edge_index.shape[1]
    hidden = W1.shape[1]
    z_dim = W2.shape[1]
    n_tot = n_edges + N_NODES
    per_tile = -(-n_tot // (NW * CHUNK * NB)) * NB
    m0, m1 = _split(per_tile)
    n_rows = NS * (m0 + m1) + max(0, m0 - m1)
    e_pad = n_rows * CHUNK

    loop = jnp.arange(N_NODES, dtype=jnp.int32)
    pad_s = jnp.zeros((e_pad - n_tot,), jnp.int32)
    pad_d = jnp.full((e_pad - n_tot,), N_NODES, jnp.int32)
    src2 = jnp.concatenate([edge_index[0], loop, pad_s]).reshape(n_rows, CHUNK)
    dst2 = jnp.concatenate([edge_index[1], loop, pad_d]).reshape(n_rows, CHUNK)

    degp = _make_deg_call(m0, m1)(dst2).reshape(NW, NPAD)

    g1 = pl.pallas_call(
        _tc1_body,
        out_shape=jax.ShapeDtypeStruct((N_NODES, hidden), jnp.float32),
    )(x, W1, degp)

    agg1 = _make_agg_call(m0, m1, hidden)(g1, src2, dst2)
    agg1 = agg1.reshape(NC, NPAD, hidden)

    g2 = pl.pallas_call(
        _tc2_body,
        out_shape=jax.ShapeDtypeStruct((N_NODES, z_dim), jnp.float32),
    )(agg1, degp, b1.reshape(1, hidden), W2)

    agg2 = _make_agg_call(m0, m1, z_dim)(g2, src2, dst2)
    agg2 = agg2.reshape(NC, NPAD, z_dim)

    z = pl.pallas_call(
        _tc3_body,
        out_shape=jax.ShapeDtypeStruct((N_NODES, z_dim), jnp.float32),
    )(agg2, degp, b2.reshape(1, z_dim))

    return z
